# baseline (speedup 1.0000x reference)
import functools

import numpy as np
import jax
import jax.numpy as jnp
from jax import lax
from jax.experimental import pallas as pl
from jax.experimental.pallas import tpu as pltpu
from jax.experimental.pallas import tpu_sc as plsc

_N = 10000
_E = 320000
_D = 128
_NUM_MASK = 3000
_NUM_REMASK = 5000
_ROUNDS = 3


_MASK_CACHE = []


def _mask_consts():
    def build():
        kperm = jax.random.key(42)
        perm0 = jax.random.permutation(jax.random.fold_in(kperm, 0), _N)
        m = jnp.zeros((_N, 1), jnp.float32).at[perm0[:_NUM_MASK]].set(1.0)
        cnt = jnp.zeros((_N,), jnp.float32)
        for r in range(_ROUNDS):
            p = jax.random.permutation(jax.random.fold_in(kperm, r + 1), _N)
            cnt = cnt.at[p[:_NUM_REMASK]].add(1.0)
        alpha = ((_ROUNDS - cnt) / _ROUNDS).reshape(_N, 1)
        beta = (cnt / _ROUNDS).reshape(_N, 1)
        return m, alpha, beta

    if not _MASK_CACHE:
        try:
            with jax.ensure_compile_time_eval():
                _MASK_CACHE.append(tuple(np.asarray(v) for v in build()))
        except Exception:
            return build()
    m, alpha, beta = _MASK_CACHE[0]
    return jnp.asarray(m), jnp.asarray(alpha), jnp.asarray(beta)


_NCORES = 2
_NSUB = 16
_NW = _NCORES * _NSUB
_CHUNK = 128
_NCHUNKS = _E // _CHUNK
_GC = 16
_NGROUPS = (_NCHUNKS + _GC - 1) // _GC
_PCHUNKS = _NGROUPS * _GC
_GITERS = (_NGROUPS + _NW - 1) // _NW
_NBUF = 2
_TILE_ROWS = 624
_WB_CHUNKS = ((0, 128), (128, 128), (256, 128), (384, 128), (512, 112))
_WB_LAST = (624, 16)

_sc_mesh = plsc.VectorSubcoreMesh(core_axis_name="c", subcore_axis_name="s")


def _zero_vmem_2d(ref, nrows):
    def body(i, _):
        r = i // (_D // 16)
        col = (i % (_D // 16)) * 16
        ref[r, pl.ds(col, 16)] = jnp.zeros((16,), jnp.float32)
        return 0
    lax.fori_loop(0, nrows * (_D // 16), body, 0)


@functools.partial(
    pl.kernel,
    out_type=[jax.ShapeDtypeStruct((_NCORES * _N,), jnp.float32),
              jax.ShapeDtypeStruct((_NCORES * _N,), jnp.float32)],
    mesh=_sc_mesh,
    scratch_types=[
        pltpu.VMEM((_GC, _CHUNK), jnp.int32),
        pltpu.VMEM((_GC, _CHUNK), jnp.int32),
        pltpu.VMEM((_CHUNK,), jnp.float32),
        pltpu.VMEM((2000,), jnp.float32),
        pltpu.VMEM((_N,), jnp.float32),
        pltpu.VMEM_SHARED((_N,), jnp.float32),
        pltpu.VMEM_SHARED((_N,), jnp.float32),
        pltpu.SemaphoreType.DMA,
    ],
)
def _degrees_sc(src2d_hbm, dst2d_hbm, osrc_hbm, odst_hbm,
                sidx, didx, ones_v, zbuf, stage_v, hsrc, hdst, sem):
    cid = lax.axis_index("c")
    sid = lax.axis_index("s")
    w = sid * _NCORES + cid

    def fill(i, _):
        ones_v[pl.ds(i * 16, 16)] = jnp.ones((16,), jnp.float32)
        return 0
    lax.fori_loop(0, _CHUNK // 16, fill, 0)

    def zfill(i, _):
        zbuf[pl.ds(i * 16, 16)] = jnp.zeros((16,), jnp.float32)
        return 0
    lax.fori_loop(0, 2000 // 16, zfill, 0)

    @pl.when(sid == 0)
    def _():
        for k in range(5):
            pltpu.sync_copy(zbuf, hsrc.at[pl.ds(k * 2000, 2000)])

    @pl.when(sid == 1)
    def _():
        for k in range(5):
            pltpu.sync_copy(zbuf, hdst.at[pl.ds(k * 2000, 2000)])

    plsc.subcore_barrier()

    def body(t, _):
        g = w + _NW * t
        base_c = g * _GC

        @pl.when(g < _NGROUPS)
        def _():
            pltpu.sync_copy(src2d_hbm.at[pl.ds(g * _GC, _GC), :], sidx)
            pltpu.sync_copy(dst2d_hbm.at[pl.ds(g * _GC, _GC), :], didx)
            for j in range(_GC):
                @pl.when(base_c + j < _NCHUNKS)
                def _(j=j):
                    pltpu.async_copy(ones_v, hsrc.at[sidx.at[j]], sem, add=True)
                    pltpu.async_copy(ones_v, hdst.at[didx.at[j]], sem, add=True)
            for j in range(_GC):
                @pl.when(base_c + j < _NCHUNKS)
                def _(j=j):
                    pltpu.make_async_copy(ones_v, hsrc.at[sidx.at[j]], sem).wait()
                    pltpu.make_async_copy(ones_v, hdst.at[didx.at[j]], sem).wait()
        return 0
    lax.fori_loop(0, _GITERS, body, 0)

    plsc.subcore_barrier()

    @pl.when(sid == 0)
    def _():
        pltpu.sync_copy(hsrc, stage_v)
        pltpu.sync_copy(stage_v, osrc_hbm.at[pl.ds(cid * _N, _N)])

    @pl.when(sid == 1)
    def _():
        pltpu.sync_copy(hdst, stage_v)
        pltpu.sync_copy(stage_v, odst_hbm.at[pl.ds(cid * _N, _N)])


@functools.partial(
    pl.kernel,
    out_type=jax.ShapeDtypeStruct((_NCORES * _N, _D), jnp.float32),
    mesh=_sc_mesh,
    scratch_types=[
        pltpu.VMEM((_GC, _CHUNK), jnp.int32),
        pltpu.VMEM((_GC, _CHUNK), jnp.int32),
        pltpu.VMEM((_CHUNK, _D), jnp.float32),
        pltpu.VMEM((_CHUNK, _D), jnp.float32),
        pltpu.VMEM_SHARED((_N, _D), jnp.float32),
        pltpu.SemaphoreType.DMA,
        pltpu.SemaphoreType.DMA,
        pltpu.SemaphoreType.DMA,
        pltpu.SemaphoreType.DMA,
    ],
)
def _conv_sc(src2d_hbm, dst2d_hbm, hs_hbm, out_hbm,
             sidx, didx, rows0, rows1, acc, gsem0, gsem1, ssem0, ssem1):
    cid = lax.axis_index("c")
    sid = lax.axis_index("s")
    w = sid * _NCORES + cid
    rows = (rows0, rows1)
    gsems = (gsem0, gsem1)
    ssems = (ssem0, ssem1)

    _zero_vmem_2d(rows0, _CHUNK)
    base_row = sid * _TILE_ROWS
    for off, nr in _WB_CHUNKS:
        pltpu.sync_copy(rows0.at[pl.ds(0, nr), :],
                        acc.at[pl.ds(base_row + off, nr), :])

    @pl.when(sid == _NSUB - 1)
    def _():
        off, nr = _WB_LAST
        pltpu.sync_copy(rows0.at[pl.ds(0, nr), :],
                        acc.at[pl.ds(base_row + off, nr), :])
    plsc.subcore_barrier()

    def body(t, _):
        g = w + _NW * t
        base_c = g * _GC

        @pl.when(g < _NGROUPS)
        def _():
            pltpu.sync_copy(src2d_hbm.at[pl.ds(g * _GC, _GC), :], sidx)
            pltpu.sync_copy(dst2d_hbm.at[pl.ds(g * _GC, _GC), :], didx)
            for j in range(_NBUF):
                @pl.when(base_c + j < _NCHUNKS)
                def _(j=j):
                    pltpu.async_copy(hs_hbm.at[sidx.at[j]], rows[j], gsems[j])
            for j in range(_GC):
                b = j % _NBUF
                cj = base_c + j < _NCHUNKS

                @pl.when(cj)
                def _(j=j, b=b):
                    pltpu.make_async_copy(hs_hbm.at[sidx.at[j]], rows[b],
                                          gsems[b]).wait()
                    pltpu.async_copy(rows[b], acc.at[didx.at[j]], ssems[b],
                                     add=True)
                if j + _NBUF < _GC:
                    cj2 = base_c + j + _NBUF < _NCHUNKS

                    @pl.when(cj2)
                    def _(j=j, b=b):
                        pltpu.make_async_copy(rows[b], acc.at[didx.at[j]],
                                              ssems[b]).wait()
                        pltpu.async_copy(hs_hbm.at[sidx.at[j + _NBUF]],
                                         rows[b], gsems[b])

                    @pl.when(cj & jnp.logical_not(cj2))
                    def _(j=j, b=b):
                        pltpu.make_async_copy(rows[b], acc.at[didx.at[j]],
                                              ssems[b]).wait()
                else:
                    @pl.when(cj)
                    def _(j=j, b=b):
                        pltpu.make_async_copy(rows[b], acc.at[didx.at[j]],
                                              ssems[b]).wait()
        return 0
    lax.fori_loop(0, _GITERS, body, 0)

    plsc.subcore_barrier()
    for off, nr in _WB_CHUNKS:
        pltpu.sync_copy(acc.at[pl.ds(base_row + off, nr), :],
                        out_hbm.at[pl.ds(cid * _N + base_row + off, nr), :])

    @pl.when(sid == _NSUB - 1)
    def _():
        off, nr = _WB_LAST
        pltpu.sync_copy(acc.at[pl.ds(base_row + off, nr), :],
                        out_hbm.at[pl.ds(cid * _N + base_row + off, nr), :])


_BR = 1000
_G = _N // _BR


def _mask_tc_body(x_ref, ps0_ref, ps1_ref, pd0_ref, pd1_ref, m_ref, tok_ref,
                  o_ref, io_ref, ii_ref, sm_ref):
    io = lax.rsqrt(jnp.maximum(ps0_ref[...] + ps1_ref[...], 1.0))
    ii = lax.rsqrt(jnp.maximum(pd0_ref[...] + pd1_ref[...], 1.0))
    m = m_ref[...]
    o_ref[...] = (x_ref[...] * (1.0 - m) + m * tok_ref[...]) * io
    io_ref[...] = io
    ii_ref[...] = ii
    sm_ref[...] = io * ii


def _mask_tc(x, osrc, odst, m, tok):
    col = pl.BlockSpec((_BR, 1), lambda i: (i, 0))
    col2 = pl.BlockSpec((_BR, 1), lambda i: (i + _G, 0))
    return pl.pallas_call(
        _mask_tc_body,
        grid=(_G,),
        in_specs=[
            pl.BlockSpec((_BR, _D), lambda i: (i, 0)),
            col, col2, col, col2, col,
            pl.BlockSpec((1, _D), lambda i: (0, 0)),
        ],
        out_specs=[pl.BlockSpec((_BR, _D), lambda i: (i, 0)), col, col, col],
        out_shape=[jax.ShapeDtypeStruct((_N, _D), jnp.float32)]
        + [jax.ShapeDtypeStruct((_N, 1), jnp.float32)] * 3,
    )(x, osrc, osrc, odst, odst, m, tok)


def _mid_tc_body(pa_ref, pb_ref, sm_ref, hs_ref, agg_ref):
    agg = pa_ref[...] + pb_ref[...]
    agg_ref[...] = agg
    hs_ref[...] = agg * sm_ref[...]


def _mid_tc(parts, sm):
    return pl.pallas_call(
        _mid_tc_body,
        grid=(_G,),
        in_specs=[
            pl.BlockSpec((_BR, _D), lambda i: (i, 0)),
            pl.BlockSpec((_BR, _D), lambda i: (i + _G, 0)),
            pl.BlockSpec((_BR, 1), lambda i: (i, 0)),
        ],
        out_specs=[pl.BlockSpec((_BR, _D), lambda i: (i, 0))] * 2,
        out_shape=[jax.ShapeDtypeStruct((_N, _D), jnp.float32)] * 2,
    )(parts, parts, sm)


def _dec_tc_body(a1_ref, pa_ref, pb_ref, ii_ref, io_ref, al_ref, be_ref,
                 w_ref, tok_ref, o_ref):
    agg2 = pa_ref[...] + pb_ref[...]
    es = 0.5 * ii_ref[...] * (a1_ref[...] + agg2)
    orep = lax.dot_general(es, w_ref[...], (((1,), (1,)), ((), ())),
                           preferred_element_type=jnp.float32)
    repm = al_ref[...] * orep + be_ref[...] * tok_ref[...]
    o_ref[...] = repm * io_ref[...]


def _dec_tc(agg1, parts2, ii, io, al, be, w_e2d, dec_tok):
    return pl.pallas_call(
        _dec_tc_body,
        grid=(_G,),
        in_specs=[
            pl.BlockSpec((_BR, _D), lambda i: (i, 0)),
            pl.BlockSpec((_BR, _D), lambda i: (i, 0)),
            pl.BlockSpec((_BR, _D), lambda i: (i + _G, 0)),
            pl.BlockSpec((_BR, 1), lambda i: (i, 0)),
            pl.BlockSpec((_BR, 1), lambda i: (i, 0)),
            pl.BlockSpec((_BR, 1), lambda i: (i, 0)),
            pl.BlockSpec((_BR, 1), lambda i: (i, 0)),
            pl.BlockSpec((_D, _D), lambda i: (0, 0)),
            pl.BlockSpec((1, _D), lambda i: (0, 0)),
        ],
        out_specs=pl.BlockSpec((_BR, _D), lambda i: (i, 0)),
        out_shape=jax.ShapeDtypeStruct((_N, _D), jnp.float32),
    )(agg1, parts2, parts2, ii, io, al, be, w_e2d, dec_tok)


def _out_tc_body(pa_ref, pb_ref, ii_ref, w_ref, o_ref):
    ds = (pa_ref[...] + pb_ref[...]) * ii_ref[...]
    o_ref[...] = lax.dot_general(ds, w_ref[...], (((1,), (1,)), ((), ())),
                                 preferred_element_type=jnp.float32)


def _out_tc(parts3, ii, w_d2c):
    return pl.pallas_call(
        _out_tc_body,
        grid=(_G,),
        in_specs=[
            pl.BlockSpec((_BR, _D), lambda i: (i, 0)),
            pl.BlockSpec((_BR, _D), lambda i: (i + _G, 0)),
            pl.BlockSpec((_BR, 1), lambda i: (i, 0)),
            pl.BlockSpec((_D, _D), lambda i: (0, 0)),
        ],
        out_specs=pl.BlockSpec((_BR, _D), lambda i: (i, 0)),
        out_shape=jax.ShapeDtypeStruct((_N, _D), jnp.float32),
    )(parts3, parts3, ii, w_d2c)


def kernel(x, edge_index, enc_mask_token, dec_mask_token, W_e2d, W_d2c):
    pad = jnp.zeros((2, _PCHUNKS * _CHUNK - _E), edge_index.dtype)
    ep = jnp.concatenate([edge_index, pad], axis=1).reshape(2, _PCHUNKS, _CHUNK)
    src = ep[0]
    dst = ep[1]
    m, alpha, beta = _mask_consts()

    osrc, odst = _degrees_sc(src, dst)
    hs0, io, ii, sm = _mask_tc(x, osrc.reshape(_NCORES * _N, 1),
                               odst.reshape(_NCORES * _N, 1), m, enc_mask_token)
    parts1 = _conv_sc(src, dst, hs0)
    hs1, agg1 = _mid_tc(parts1, sm)
    parts2 = _conv_sc(src, dst, hs1)
    hs2 = _dec_tc(agg1, parts2, ii, io, alpha, beta, W_e2d, dec_mask_token)
    parts3 = _conv_sc(src, dst, hs2)
    return _out_tc(parts3, ii, W_d2c)


# speedup vs baseline: 1.0358x; 1.0358x over previous
import functools

import numpy as np
import jax
import jax.numpy as jnp
from jax import lax
from jax.experimental import pallas as pl
from jax.experimental.pallas import tpu as pltpu
from jax.experimental.pallas import tpu_sc as plsc

_N = 10000
_E = 320000
_D = 128
_NUM_MASK = 3000
_NUM_REMASK = 5000
_ROUNDS = 3


_MASK_CACHE = []


def _mask_consts():
    def build():
        kperm = jax.random.key(42)
        perm0 = jax.random.permutation(jax.random.fold_in(kperm, 0), _N)
        m = jnp.zeros((_N, 1), jnp.float32).at[perm0[:_NUM_MASK]].set(1.0)
        cnt = jnp.zeros((_N,), jnp.float32)
        for r in range(_ROUNDS):
            p = jax.random.permutation(jax.random.fold_in(kperm, r + 1), _N)
            cnt = cnt.at[p[:_NUM_REMASK]].add(1.0)
        alpha = ((_ROUNDS - cnt) / _ROUNDS).reshape(_N, 1)
        beta = (cnt / _ROUNDS).reshape(_N, 1)
        return m, alpha, beta

    if not _MASK_CACHE:
        try:
            with jax.ensure_compile_time_eval():
                _MASK_CACHE.append(tuple(np.asarray(v) for v in build()))
        except Exception:
            return build()
    m, alpha, beta = _MASK_CACHE[0]
    return jnp.asarray(m), jnp.asarray(alpha), jnp.asarray(beta)


_NCORES = 2
_NSUB = 16
_NW = _NCORES * _NSUB
_CHUNK = 128
_NCHUNKS = _E // _CHUNK
_GC = 16
_NGROUPS = (_NCHUNKS + _GC - 1) // _GC
_PCHUNKS = _NGROUPS * _GC
_GITERS = (_NGROUPS + _NW - 1) // _NW
_NBUF = 2
_TILE_ROWS = 624
_WB_CHUNKS = ((0, 128), (128, 128), (256, 128), (384, 128), (512, 112))
_WB_LAST = (624, 16)

_sc_mesh = plsc.VectorSubcoreMesh(core_axis_name="c", subcore_axis_name="s")


def _zero_vmem_2d(ref, nrows):
    def body(i, _):
        r = i // (_D // 16)
        col = (i % (_D // 16)) * 16
        ref[r, pl.ds(col, 16)] = jnp.zeros((16,), jnp.float32)
        return 0
    lax.fori_loop(0, nrows * (_D // 16), body, 0)


@functools.partial(
    pl.kernel,
    out_type=[jax.ShapeDtypeStruct((_NCORES * _N,), jnp.float32),
              jax.ShapeDtypeStruct((_NCORES * _N,), jnp.float32)],
    mesh=_sc_mesh,
    scratch_types=[
        pltpu.VMEM((_GC, _CHUNK), jnp.int32),
        pltpu.VMEM((_GC, _CHUNK), jnp.int32),
        pltpu.VMEM((_CHUNK,), jnp.float32),
        pltpu.VMEM((2000,), jnp.float32),
        pltpu.VMEM((_N,), jnp.float32),
        pltpu.VMEM_SHARED((_N,), jnp.float32),
        pltpu.VMEM_SHARED((_N,), jnp.float32),
        pltpu.SemaphoreType.DMA,
    ],
)
def _degrees_sc(src2d_hbm, dst2d_hbm, osrc_hbm, odst_hbm,
                sidx, didx, ones_v, zbuf, stage_v, hsrc, hdst, sem):
    cid = lax.axis_index("c")
    sid = lax.axis_index("s")
    w = sid * _NCORES + cid

    def fill(i, _):
        ones_v[pl.ds(i * 16, 16)] = jnp.ones((16,), jnp.float32)
        return 0
    lax.fori_loop(0, _CHUNK // 16, fill, 0)

    def zfill(i, _):
        zbuf[pl.ds(i * 16, 16)] = jnp.zeros((16,), jnp.float32)
        return 0
    lax.fori_loop(0, 2000 // 16, zfill, 0)

    @pl.when(sid == 0)
    def _():
        for k in range(5):
            pltpu.sync_copy(zbuf, hsrc.at[pl.ds(k * 2000, 2000)])

    @pl.when(sid == 1)
    def _():
        for k in range(5):
            pltpu.sync_copy(zbuf, hdst.at[pl.ds(k * 2000, 2000)])

    plsc.subcore_barrier()

    def body(t, _):
        g = w + _NW * t
        base_c = g * _GC

        @pl.when(g < _NGROUPS)
        def _():
            pltpu.sync_copy(src2d_hbm.at[pl.ds(g * _GC, _GC), :], sidx)
            pltpu.sync_copy(dst2d_hbm.at[pl.ds(g * _GC, _GC), :], didx)
            for j in range(_GC):
                @pl.when(base_c + j < _NCHUNKS)
                def _(j=j):
                    pltpu.async_copy(ones_v, hsrc.at[sidx.at[j]], sem, add=True)
                    pltpu.async_copy(ones_v, hdst.at[didx.at[j]], sem, add=True)
            for j in range(_GC):
                @pl.when(base_c + j < _NCHUNKS)
                def _(j=j):
                    pltpu.make_async_copy(ones_v, hsrc.at[sidx.at[j]], sem).wait()
                    pltpu.make_async_copy(ones_v, hdst.at[didx.at[j]], sem).wait()
        return 0
    lax.fori_loop(0, _GITERS, body, 0)

    plsc.subcore_barrier()

    @pl.when(sid == 0)
    def _():
        pltpu.sync_copy(hsrc, stage_v)
        pltpu.sync_copy(stage_v, osrc_hbm.at[pl.ds(cid * _N, _N)])

    @pl.when(sid == 1)
    def _():
        pltpu.sync_copy(hdst, stage_v)
        pltpu.sync_copy(stage_v, odst_hbm.at[pl.ds(cid * _N, _N)])


@functools.partial(
    pl.kernel,
    out_type=jax.ShapeDtypeStruct((_NCORES * _N, _D), jnp.float32),
    mesh=_sc_mesh,
    scratch_types=[
        pltpu.VMEM((_GC, _CHUNK), jnp.int32),
        pltpu.VMEM((_GC, _CHUNK), jnp.int32),
        pltpu.VMEM((_CHUNK, _D), jnp.float32),
        pltpu.VMEM((_CHUNK, _D), jnp.float32),
        pltpu.VMEM_SHARED((_N, _D), jnp.float32),
        pltpu.SemaphoreType.DMA,
        pltpu.SemaphoreType.DMA,
        pltpu.SemaphoreType.DMA,
        pltpu.SemaphoreType.DMA,
    ],
)
def _conv_sc(src2d_hbm, dst2d_hbm, hs_hbm, out_hbm,
             sidx, didx, rows0, rows1, acc, gsem0, gsem1, ssem0, ssem1):
    cid = lax.axis_index("c")
    sid = lax.axis_index("s")
    w = sid * _NCORES + cid
    rows = (rows0, rows1)
    gsems = (gsem0, gsem1)
    ssems = (ssem0, ssem1)

    _zero_vmem_2d(rows0, _CHUNK)
    base_row = sid * _TILE_ROWS
    for off, nr in _WB_CHUNKS:
        pltpu.sync_copy(rows0.at[pl.ds(0, nr), :],
                        acc.at[pl.ds(base_row + off, nr), :])

    @pl.when(sid == _NSUB - 1)
    def _():
        off, nr = _WB_LAST
        pltpu.sync_copy(rows0.at[pl.ds(0, nr), :],
                        acc.at[pl.ds(base_row + off, nr), :])
    plsc.subcore_barrier()

    def body(t, _):
        g = w + _NW * t
        base_c = g * _GC

        @pl.when(g < _NGROUPS)
        def _():
            pltpu.sync_copy(src2d_hbm.at[pl.ds(g * _GC, _GC), :], sidx)
            pltpu.sync_copy(dst2d_hbm.at[pl.ds(g * _GC, _GC), :], didx)
            for j in range(_NBUF):
                @pl.when(base_c + j < _NCHUNKS)
                def _(j=j):
                    pltpu.async_copy(hs_hbm.at[sidx.at[j]], rows[j], gsems[j])
            for j in range(_GC):
                b = j % _NBUF
                cj = base_c + j < _NCHUNKS

                @pl.when(cj)
                def _(j=j, b=b):
                    pltpu.make_async_copy(hs_hbm.at[sidx.at[j]], rows[b],
                                          gsems[b]).wait()
                    pltpu.async_copy(rows[b], acc.at[didx.at[j]], ssems[b],
                                     add=True)
                if j + _NBUF < _GC:
                    cj2 = base_c + j + _NBUF < _NCHUNKS

                    @pl.when(cj2)
                    def _(j=j, b=b):
                        pltpu.make_async_copy(rows[b], acc.at[didx.at[j]],
                                              ssems[b]).wait()
                        pltpu.async_copy(hs_hbm.at[sidx.at[j + _NBUF]],
                                         rows[b], gsems[b])

                    @pl.when(cj & jnp.logical_not(cj2))
                    def _(j=j, b=b):
                        pltpu.make_async_copy(rows[b], acc.at[didx.at[j]],
                                              ssems[b]).wait()
                else:
                    @pl.when(cj)
                    def _(j=j, b=b):
                        pltpu.make_async_copy(rows[b], acc.at[didx.at[j]],
                                              ssems[b]).wait()
        return 0
    lax.fori_loop(0, _GITERS, body, 0)

    plsc.subcore_barrier()
    for off, nr in _WB_CHUNKS:
        pltpu.sync_copy(acc.at[pl.ds(base_row + off, nr), :],
                        out_hbm.at[pl.ds(cid * _N + base_row + off, nr), :])

    @pl.when(sid == _NSUB - 1)
    def _():
        off, nr = _WB_LAST
        pltpu.sync_copy(acc.at[pl.ds(base_row + off, nr), :],
                        out_hbm.at[pl.ds(cid * _N + base_row + off, nr), :])


_PAD = 10240
_BR = 1000
_G = _N // _BR


def _deg_tc_body(ps_ref, pd_ref, io_ref, ii_ref, sm_ref):
    ds = jnp.maximum(ps_ref[0:1, :] + ps_ref[1:2, :], 1.0)
    dd = jnp.maximum(pd_ref[0:1, :] + pd_ref[1:2, :], 1.0)
    io = lax.rsqrt(ds)
    ii = lax.rsqrt(dd)
    io_ref[...] = io
    ii_ref[...] = ii
    sm_ref[...] = io * ii


def _deg_tc(ps, pd):
    return pl.pallas_call(
        _deg_tc_body,
        out_shape=[jax.ShapeDtypeStruct((1, _PAD), jnp.float32)] * 3,
    )(ps, pd)


def _mask_tc_body(x_ref, io_ref, m_ref, tok_ref, o_ref):
    m = m_ref[...]
    o_ref[...] = (x_ref[...] * (1.0 - m) + m * tok_ref[...]) * io_ref[...]


def _mask_tc(x, io, m, tok):
    return pl.pallas_call(
        _mask_tc_body,
        grid=(_G,),
        in_specs=[
            pl.BlockSpec((_BR, _D), lambda i: (i, 0)),
            pl.BlockSpec((_BR, 1), lambda i: (i, 0)),
            pl.BlockSpec((_BR, 1), lambda i: (i, 0)),
            pl.BlockSpec((1, _D), lambda i: (0, 0)),
        ],
        out_specs=pl.BlockSpec((_BR, _D), lambda i: (i, 0)),
        out_shape=jax.ShapeDtypeStruct((_N, _D), jnp.float32),
    )(x, io, m, tok)


def _mid_tc_body(pa_ref, pb_ref, sm_ref, hs_ref, agg_ref):
    agg = pa_ref[...] + pb_ref[...]
    agg_ref[...] = agg
    hs_ref[...] = agg * sm_ref[...]


def _mid_tc(parts, sm):
    return pl.pallas_call(
        _mid_tc_body,
        grid=(_G,),
        in_specs=[
            pl.BlockSpec((_BR, _D), lambda i: (i, 0)),
            pl.BlockSpec((_BR, _D), lambda i: (i + _G, 0)),
            pl.BlockSpec((_BR, 1), lambda i: (i, 0)),
        ],
        out_specs=[pl.BlockSpec((_BR, _D), lambda i: (i, 0))] * 2,
        out_shape=[jax.ShapeDtypeStruct((_N, _D), jnp.float32)] * 2,
    )(parts, parts, sm)


def _dec_tc_body(a1_ref, pa_ref, pb_ref, ii_ref, io_ref, al_ref, be_ref,
                 w_ref, tok_ref, o_ref):
    agg2 = pa_ref[...] + pb_ref[...]
    es = 0.5 * ii_ref[...] * (a1_ref[...] + agg2)
    orep = lax.dot_general(es, w_ref[...], (((1,), (1,)), ((), ())),
                           preferred_element_type=jnp.float32)
    repm = al_ref[...] * orep + be_ref[...] * tok_ref[...]
    o_ref[...] = repm * io_ref[...]


def _dec_tc(agg1, parts2, ii, io, al, be, w_e2d, dec_tok):
    return pl.pallas_call(
        _dec_tc_body,
        grid=(_G,),
        in_specs=[
            pl.BlockSpec((_BR, _D), lambda i: (i, 0)),
            pl.BlockSpec((_BR, _D), lambda i: (i, 0)),
            pl.BlockSpec((_BR, _D), lambda i: (i + _G, 0)),
            pl.BlockSpec((_BR, 1), lambda i: (i, 0)),
            pl.BlockSpec((_BR, 1), lambda i: (i, 0)),
            pl.BlockSpec((_BR, 1), lambda i: (i, 0)),
            pl.BlockSpec((_BR, 1), lambda i: (i, 0)),
            pl.BlockSpec((_D, _D), lambda i: (0, 0)),
            pl.BlockSpec((1, _D), lambda i: (0, 0)),
        ],
        out_specs=pl.BlockSpec((_BR, _D), lambda i: (i, 0)),
        out_shape=jax.ShapeDtypeStruct((_N, _D), jnp.float32),
    )(agg1, parts2, parts2, ii, io, al, be, w_e2d, dec_tok)


def _out_tc_body(pa_ref, pb_ref, ii_ref, w_ref, o_ref):
    ds = (pa_ref[...] + pb_ref[...]) * ii_ref[...]
    o_ref[...] = lax.dot_general(ds, w_ref[...], (((1,), (1,)), ((), ())),
                                 preferred_element_type=jnp.float32)


def _out_tc(parts3, ii, w_d2c):
    return pl.pallas_call(
        _out_tc_body,
        grid=(_G,),
        in_specs=[
            pl.BlockSpec((_BR, _D), lambda i: (i, 0)),
            pl.BlockSpec((_BR, _D), lambda i: (i + _G, 0)),
            pl.BlockSpec((_BR, 1), lambda i: (i, 0)),
            pl.BlockSpec((_D, _D), lambda i: (0, 0)),
        ],
        out_specs=pl.BlockSpec((_BR, _D), lambda i: (i, 0)),
        out_shape=jax.ShapeDtypeStruct((_N, _D), jnp.float32),
    )(parts3, parts3, ii, w_d2c)


def kernel(x, edge_index, enc_mask_token, dec_mask_token, W_e2d, W_d2c):
    pad = jnp.zeros((2, _PCHUNKS * _CHUNK - _E), edge_index.dtype)
    ep = jnp.concatenate([edge_index, pad], axis=1).reshape(2, _PCHUNKS, _CHUNK)
    src = ep[0]
    dst = ep[1]
    m, alpha, beta = _mask_consts()

    osrc, odst = _degrees_sc(src, dst)
    ps = jnp.pad(osrc.reshape(_NCORES, _N), ((0, 0), (0, _PAD - _N)))
    pd = jnp.pad(odst.reshape(_NCORES, _N), ((0, 0), (0, _PAD - _N)))
    io_p, ii_p, sm_p = _deg_tc(ps, pd)
    io = io_p[0, :_N].reshape(_N, 1)
    ii = ii_p[0, :_N].reshape(_N, 1)
    sm = sm_p[0, :_N].reshape(_N, 1)

    hs0 = _mask_tc(x, io, m, enc_mask_token)
    parts1 = _conv_sc(src, dst, hs0)
    hs1, agg1 = _mid_tc(parts1, sm)
    parts2 = _conv_sc(src, dst, hs1)
    hs2 = _dec_tc(agg1, parts2, ii, io, alpha, beta, W_e2d, dec_mask_token)
    parts3 = _conv_sc(src, dst, hs2)
    return _out_tc(parts3, ii, W_d2c)
